# baseline (speedup 1.0000x reference)
import functools

import jax
import jax.numpy as jnp
from jax import lax
from jax.experimental import pallas as pl
from jax.experimental.pallas import tpu as pltpu
from jax.experimental.pallas import tpu_sc as plsc

K = 512
D = 768
ITERS = 10
RBLK = 784
NBLK = 8
P = RBLK * NBLK

PPAD = 6400
NW = 32
BPW = PPAD // NW
CHUNK = 40


def _iter_body(x_ref, xsq_ref, csq_ref, cin_ref, cout_ref, lab_ref,
               sums, counts):
    f32 = jnp.float32
    col_iota = lax.broadcasted_iota(jnp.int32, (RBLK, K), 1)
    row_iota = lax.broadcasted_iota(jnp.int32, (K, RBLK), 0)
    c = cin_ref[:, :]
    csq = csq_ref[0, :]
    sums[:, :] = jnp.zeros((K, D), f32)
    counts[:, :] = jnp.zeros((K, 1), f32)
    for blk in range(NBLK):
        xb = x_ref[blk * RBLK:(blk + 1) * RBLK, :]
        dots = lax.dot_general(xb, c, (((1,), (1,)), ((), ())),
                               preferred_element_type=f32)
        dists = (xsq_ref[blk, :][:, None] - 2.0 * dots) + csq[None, :]
        dmin = jnp.min(dists, axis=1, keepdims=True)
        lab = jnp.min(jnp.where(dists == dmin, col_iota, K), axis=1)
        lab_ref[blk, :] = lab
        et = (lab[None, :] == row_iota).astype(f32)
        sums[:, :] += lax.dot_general(et, xb, (((1,), (0,)), ((), ())),
                                      preferred_element_type=f32,
                                      precision=lax.Precision.HIGHEST)
        counts[:, :] += jnp.sum(et, axis=1, keepdims=True)
    cnt = counts[:, :]
    cout_ref[:, :] = jnp.where(cnt > 0.0,
                               sums[:, :] / jnp.maximum(cnt, 1.0), c)


_iter_call = pl.pallas_call(
    _iter_body,
    out_shape=(
        jax.ShapeDtypeStruct((K, D), jnp.float32),
        jax.ShapeDtypeStruct((NBLK, RBLK), jnp.int32),
    ),
    scratch_shapes=[
        pltpu.VMEM((K, D), jnp.float32),
        pltpu.VMEM((K, 1), jnp.float32),
    ],
)


def _table_body(c_ref, w_ref, b_ref, t_ref):
    t_ref[:, :] = lax.dot_general(
        c_ref[:, :], w_ref[:, :], (((1,), (1,)), ((), ())),
        preferred_element_type=jnp.float32) + b_ref[0, :][None, :]


_table_call = pl.pallas_call(
    _table_body,
    out_shape=jax.ShapeDtypeStruct((K, D), jnp.float32),
)


def _sc_gather(table, idx_pad):
    mesh = plsc.VectorSubcoreMesh(core_axis_name="c", subcore_axis_name="s")

    @functools.partial(
        pl.kernel, mesh=mesh,
        out_type=jax.ShapeDtypeStruct((PPAD, D), jnp.float32),
        scratch_types=[
            pltpu.VMEM((BPW,), jnp.int32),
            pltpu.VMEM((CHUNK, D), jnp.float32),
            pltpu.SemaphoreType.DMA,
        ],
    )
    def k(table_hbm, idx_hbm, out_hbm, idx_v, rows_v, sem):
        wid = lax.axis_index("s") * 2 + lax.axis_index("c")
        base = wid * BPW
        pltpu.sync_copy(idx_hbm.at[pl.ds(base, BPW)], idx_v)
        for c in range(BPW // CHUNK):
            pltpu.async_copy(
                table_hbm.at[idx_v.at[pl.ds(c * CHUNK, CHUNK)]], rows_v, sem
            ).wait()
            pltpu.sync_copy(rows_v,
                            out_hbm.at[pl.ds(base + c * CHUNK, CHUNK)])

    return k(table, idx_pad)


def kernel(x, W, b):
    B, N, Dx = x.shape
    x_flat = x.reshape(-1, Dx)
    x_sq = jnp.sum(x_flat * x_flat, axis=1, keepdims=True)
    xsq_b = x_sq.reshape(NBLK, RBLK)
    centers = x_flat[:K]
    labels = None
    for _ in range(ITERS):
        c_sq = jnp.sum(centers * centers, axis=1)
        centers, labels = _iter_call(x_flat, xsq_b, c_sq.reshape(1, K),
                                     centers)
    table = _table_call(centers, W, b.reshape(1, D))
    idx_pad = jnp.pad(labels.reshape(-1), (0, PPAD - P))
    out = _sc_gather(table, idx_pad)
    return out[:P].reshape(B, N, Dx)


# speedup vs baseline: 1.0447x; 1.0447x over previous
import functools

import jax
import jax.numpy as jnp
from jax import lax
from jax.experimental import pallas as pl
from jax.experimental.pallas import tpu as pltpu
from jax.experimental.pallas import tpu_sc as plsc

K = 512
D = 768
ITERS = 10
RBLK = 784
NBLK = 8
P = RBLK * NBLK

PPAD = 6400
NW = 32
BPW = PPAD // NW
CHUNK = 40


def _iter_body(x_ref, xsq_ref, csq_ref, cin_ref, cout_ref, lab_ref,
               sums, counts):
    f32 = jnp.float32
    col_iota = lax.broadcasted_iota(jnp.int32, (RBLK, K), 1)
    row_iota = lax.broadcasted_iota(jnp.int32, (K, RBLK), 0)
    c = cin_ref[:, :]
    csq = csq_ref[0, :]
    sums[:, :] = jnp.zeros((K, D), f32)
    counts[:, :] = jnp.zeros((K, 1), f32)
    for blk in range(NBLK):
        xb = x_ref[blk * RBLK:(blk + 1) * RBLK, :]
        dots = lax.dot_general(xb, c, (((1,), (1,)), ((), ())),
                               preferred_element_type=f32)
        dists = (xsq_ref[blk, :][:, None] - 2.0 * dots) + csq[None, :]
        dmin = jnp.min(dists, axis=1, keepdims=True)
        lab = jnp.min(jnp.where(dists == dmin, col_iota, K), axis=1)
        lab_ref[blk, :] = lab
        et = (lab[None, :] == row_iota).astype(f32)
        x_hi = xb.astype(jnp.bfloat16).astype(f32)
        r1 = xb - x_hi
        x_mid = r1.astype(jnp.bfloat16).astype(f32)
        x_lo = r1 - x_mid
        dn = (((1,), (0,)), ((), ()))
        acc = lax.dot_general(et, x_lo, dn, preferred_element_type=f32)
        acc += lax.dot_general(et, x_mid, dn, preferred_element_type=f32)
        acc += lax.dot_general(et, x_hi, dn, preferred_element_type=f32)
        sums[:, :] += acc
        counts[:, :] += jnp.sum(et, axis=1, keepdims=True)
    cnt = counts[:, :]
    cout_ref[:, :] = jnp.where(cnt > 0.0,
                               sums[:, :] / jnp.maximum(cnt, 1.0), c)


_iter_call = pl.pallas_call(
    _iter_body,
    out_shape=(
        jax.ShapeDtypeStruct((K, D), jnp.float32),
        jax.ShapeDtypeStruct((NBLK, RBLK), jnp.int32),
    ),
    scratch_shapes=[
        pltpu.VMEM((K, D), jnp.float32),
        pltpu.VMEM((K, 1), jnp.float32),
    ],
)


def _table_body(c_ref, w_ref, b_ref, t_ref):
    t_ref[:, :] = lax.dot_general(
        c_ref[:, :], w_ref[:, :], (((1,), (1,)), ((), ())),
        preferred_element_type=jnp.float32) + b_ref[0, :][None, :]


_table_call = pl.pallas_call(
    _table_body,
    out_shape=jax.ShapeDtypeStruct((K, D), jnp.float32),
)


def _sc_gather(table, idx_pad):
    mesh = plsc.VectorSubcoreMesh(core_axis_name="c", subcore_axis_name="s")

    @functools.partial(
        pl.kernel, mesh=mesh,
        out_type=jax.ShapeDtypeStruct((PPAD, D), jnp.float32),
        scratch_types=[
            pltpu.VMEM((BPW,), jnp.int32),
            pltpu.VMEM((CHUNK, D), jnp.float32),
            pltpu.SemaphoreType.DMA,
        ],
    )
    def k(table_hbm, idx_hbm, out_hbm, idx_v, rows_v, sem):
        wid = lax.axis_index("s") * 2 + lax.axis_index("c")
        base = wid * BPW
        pltpu.sync_copy(idx_hbm.at[pl.ds(base, BPW)], idx_v)
        for c in range(BPW // CHUNK):
            pltpu.async_copy(
                table_hbm.at[idx_v.at[pl.ds(c * CHUNK, CHUNK)]], rows_v, sem
            ).wait()
            pltpu.sync_copy(rows_v,
                            out_hbm.at[pl.ds(base + c * CHUNK, CHUNK)])

    return k(table, idx_pad)


def kernel(x, W, b):
    B, N, Dx = x.shape
    x_flat = x.reshape(-1, Dx)
    x_sq = jnp.sum(x_flat * x_flat, axis=1, keepdims=True)
    xsq_b = x_sq.reshape(NBLK, RBLK)
    centers = x_flat[:K]
    labels = None
    for _ in range(ITERS):
        c_sq = jnp.sum(centers * centers, axis=1)
        centers, labels = _iter_call(x_flat, xsq_b, c_sq.reshape(1, K),
                                     centers)
    table = _table_call(centers, W, b.reshape(1, D))
    idx_pad = jnp.pad(labels.reshape(-1), (0, PPAD - P))
    out = _sc_gather(table, idx_pad)
    return out[:P].reshape(B, N, Dx)
